# baseline (speedup 1.0000x reference)
import jax
import jax.numpy as jnp
import numpy as np
from jax.experimental import pallas as pl
from jax.experimental.pallas import tpu as pltpu

_WIN = 11
_EGO = 33
_NCLS = 4
_LSTM_IN = 726
_IN_PAD = 768
_HPAD = 1024
_NH = 512
_M = 9
_OUT = _WIN * _WIN * _NCLS


def _perm_consts():
    a = np.arange(512)[:, None]
    b = np.arange(_IN_PAD)[None, :]
    tgt = (a % 4) * 121 + 11 * (a // 44) + (a % 44) // 4
    p2 = ((b == tgt) & (a < 484))
    p = np.arange(128)[:, None]
    q4 = ((b == 484 + p) & (p < 121))
    q5 = ((b == 605 + p) & (p < 121))
    wi = np.arange(11)[:, None]
    c = np.arange(512)[None, :]
    tmask = ((c // 44 == wi) & (c < 484))
    return tuple(jnp.asarray(x.astype(np.float32), dtype=jnp.bfloat16)
                 for x in (tmask, p2, q4, q5))


def _lstm2_kernel(cd_ref, mo_ref, gr_ref, hp0_ref, hp1_ref, c0p_ref, b0_ref,
                  tm_ref, p2_ref, q4_ref, q5_ref, w0_hbm, w1_hbm,
                  h0_ref, c0_ref, part_ref,
                  w0_buf, w1a_buf, w1b_buf, sem0, sem1):
    n = pl.program_id(0)

    for g in range(4):
        pltpu.make_async_copy(w0_hbm.at[2 * g + n], w0_buf.at[g],
                              sem0.at[g]).start()
    pltpu.make_async_copy(w1_hbm.at[:, pl.ds(n * _NH, _NH), :],
                          w1a_buf, sem1.at[0]).start()
    pltpu.make_async_copy(w1_hbm.at[:, pl.ds(_HPAD + n * _NH, _NH), :],
                          w1b_buf, sem1.at[1]).start()

    s0 = jnp.clip(34 + cd_ref[0] + mo_ref[0], 0, 67)
    s1 = jnp.clip(34 + cd_ref[1] + mo_ref[1], 0, 67)
    ones11 = jnp.ones((1, 11), jnp.bfloat16)
    tm = tm_ref[...]
    rows = []
    for w in range(9):
        i, j = w // 3, w % 3
        awin = jnp.maximum(gr_ref[i, :, j, :], 0.0).astype(jnp.bfloat16)
        atile = jnp.concatenate([awin] * 12, axis=1)[:, :512] * tm
        rows.append(jnp.dot(ones11, atile,
                            preferred_element_type=jnp.float32))
    vbig = jnp.concatenate(rows, axis=0).astype(jnp.bfloat16)

    r9 = jax.lax.broadcasted_iota(jnp.int32, (_M, 128), 0)
    c128 = jax.lax.broadcasted_iota(jnp.int32, (_M, 128), 1)
    ivec = (r9 >= 3).astype(jnp.int32) + (r9 >= 6).astype(jnp.int32)
    jvec = r9 - 3 * ivec
    wivec = jnp.zeros_like(c128)
    for t in range(1, 11):
        wivec = wivec + (c128 >= 11 * t).astype(jnp.int32)
    wjvec = c128 - 11 * wivec
    ch4 = ((s0 + 11 * ivec + wivec).astype(jnp.float32) / 100.0)
    ch5 = ((s1 + 11 * jvec + wjvec).astype(jnp.float32) / 100.0)

    d = (jnp.dot(vbig, p2_ref[...], preferred_element_type=jnp.float32)
         + jnp.dot(ch4.astype(jnp.bfloat16), q4_ref[...],
                   preferred_element_type=jnp.float32)
         + jnp.dot(ch5.astype(jnp.bfloat16), q5_ref[...],
                   preferred_element_type=jnp.float32))
    db = d.astype(jnp.bfloat16)
    hp0b = hp0_ref[0].astype(jnp.bfloat16)

    pre = []
    for g in range(4):
        pltpu.make_async_copy(w0_hbm.at[0], w0_buf.at[g], sem0.at[g]).wait()
        b = b0_ref[0, pl.ds(g * 2 * _NH + n * _NH, _NH)]
        pre.append(b + jnp.dot(db, w0_buf[g][:_IN_PAD],
                               preferred_element_type=jnp.float32)
                   + jnp.dot(hp0b, w0_buf[g][_IN_PAD:],
                             preferred_element_type=jnp.float32))
    c0 = jax.nn.sigmoid(pre[1]) * c0p_ref[0] + \
        jax.nn.sigmoid(pre[0]) * jnp.tanh(pre[2])
    h0 = jax.nn.sigmoid(pre[3]) * jnp.tanh(c0)
    h0_ref[...] = h0
    c0_ref[...] = c0

    h0b = h0.astype(jnp.bfloat16)
    hpb = hp1_ref[0].astype(jnp.bfloat16)
    pltpu.make_async_copy(w1_hbm.at[:, pl.ds(0, _NH), :], w1a_buf,
                          sem1.at[0]).wait()
    pltpu.make_async_copy(w1_hbm.at[:, pl.ds(0, _NH), :], w1b_buf,
                          sem1.at[1]).wait()
    parts = []
    for blk in range(8):
        parts.append(
            jnp.dot(h0b, w1a_buf[blk], preferred_element_type=jnp.float32)
            + jnp.dot(hpb, w1b_buf[blk], preferred_element_type=jnp.float32))
    part_ref[0] = jnp.concatenate(parts, axis=1)


def _run_lstm_pair(c_disp, motion, gr, h_all, c_all, b0, w0, w1):
    return pl.pallas_call(
        _lstm2_kernel,
        out_shape=(
            jax.ShapeDtypeStruct((_M, _HPAD), jnp.float32),
            jax.ShapeDtypeStruct((_M, _HPAD), jnp.float32),
            jax.ShapeDtypeStruct((2, _M, 8 * _NH), jnp.float32),
        ),
        grid_spec=pltpu.PrefetchScalarGridSpec(
            num_scalar_prefetch=2,
            grid=(2,),
            in_specs=[
                pl.BlockSpec((3, 11, 3, 44), lambda n, *_: (0, 0, 0, 0)),
                pl.BlockSpec((1, _M, _HPAD), lambda n, *_: (0, 0, 0)),
                pl.BlockSpec((1, _M, _NH), lambda n, *_: (1, 0, n)),
                pl.BlockSpec((1, _M, _NH), lambda n, *_: (0, 0, n)),
                pl.BlockSpec((1, 8 * _NH), lambda n, *_: (0, 0)),
                pl.BlockSpec((11, 512), lambda n, *_: (0, 0)),
                pl.BlockSpec((512, _IN_PAD), lambda n, *_: (0, 0)),
                pl.BlockSpec((128, _IN_PAD), lambda n, *_: (0, 0)),
                pl.BlockSpec((128, _IN_PAD), lambda n, *_: (0, 0)),
                pl.BlockSpec(memory_space=pl.ANY),
                pl.BlockSpec(memory_space=pl.ANY),
            ],
            out_specs=(
                pl.BlockSpec((_M, _NH), lambda n, *_: (0, n)),
                pl.BlockSpec((_M, _NH), lambda n, *_: (0, n)),
                pl.BlockSpec((1, _M, 8 * _NH), lambda n, *_: (n, 0, 0)),
            ),
            scratch_shapes=[
                pltpu.VMEM((4, _IN_PAD + _HPAD, _NH), jnp.bfloat16),
                pltpu.VMEM((8, _NH, _NH), jnp.bfloat16),
                pltpu.VMEM((8, _NH, _NH), jnp.bfloat16),
                pltpu.SemaphoreType.DMA((4,)),
                pltpu.SemaphoreType.DMA((2,)),
            ],
        ),
        compiler_params=pltpu.CompilerParams(
            dimension_semantics=("parallel",),
            vmem_limit_bytes=50 * 1024 * 1024,
        ),
    )(c_disp, motion, gr, h_all, h_all, c_all, b0, *_perm_consts(),
      w0, w1)


def _head_kernel(cd_ref, mo_ref, p_ref, b1_ref, c1p_ref, h0_ref, c0_ref,
                 w1_ref, bf1_ref, w2_ref, bf2_ref, w3_ref, bf3_ref,
                 out_ref, hs_ref, cs_ref, cdn_ref):
    pre = p_ref[0] + p_ref[1] + b1_ref[...]
    gi = jax.nn.sigmoid(pre[:, 0 * _HPAD:1 * _HPAD])
    gf = jax.nn.sigmoid(pre[:, 1 * _HPAD:2 * _HPAD])
    gg = jnp.tanh(pre[:, 2 * _HPAD:3 * _HPAD])
    go = jax.nn.sigmoid(pre[:, 3 * _HPAD:4 * _HPAD])
    c1 = gf * c1p_ref[0] + gi * gg
    h1 = go * jnp.tanh(c1)
    hs_ref[0] = h0_ref[...]
    hs_ref[1] = h1
    cs_ref[0] = c0_ref[...]
    cs_ref[1] = c1
    lane = jax.lax.broadcasted_iota(jnp.int32, (1, 2), 1)
    cdn_ref[...] = jnp.where(lane == 0, cd_ref[0] + mo_ref[0],
                             cd_ref[1] + mo_ref[1])
    t = jnp.dot(h1.astype(jnp.bfloat16), w1_ref[...],
                preferred_element_type=jnp.float32) + bf1_ref[...]
    t = jnp.maximum(t, 0.0)
    t = jnp.dot(t.astype(jnp.bfloat16), w2_ref[...],
                preferred_element_type=jnp.float32) + bf2_ref[...]
    t = jnp.maximum(t, 0.0)
    out_ref[...] = jnp.dot(t.astype(jnp.bfloat16), w3_ref[...],
                           preferred_element_type=jnp.float32) + bf3_ref[...]


def _run_head(c_disp, motion, part, b1, c_all, h0, c0,
              w1, bf1, w2, bf2, w3, bf3):
    operands = (part, b1, c_all, h0, c0, w1, bf1, w2, bf2, w3, bf3)
    in_specs = [pl.BlockSpec(op.shape, lambda i, *_, nd=op.ndim: (0,) * nd)
                for op in operands]
    in_specs[2] = pl.BlockSpec((1, _M, _HPAD), lambda i, *_: (1, 0, 0))
    return pl.pallas_call(
        _head_kernel,
        out_shape=(
            jax.ShapeDtypeStruct((_M, 512), jnp.float32),
            jax.ShapeDtypeStruct((2, _M, _HPAD), jnp.float32),
            jax.ShapeDtypeStruct((2, _M, _HPAD), jnp.float32),
            jax.ShapeDtypeStruct((1, 2), jnp.int32),
        ),
        grid_spec=pltpu.PrefetchScalarGridSpec(
            num_scalar_prefetch=2,
            grid=(1,),
            in_specs=in_specs,
            out_specs=(
                pl.BlockSpec((_M, 512), lambda i, *_: (0, 0)),
                pl.BlockSpec((2, _M, _HPAD), lambda i, *_: (0, 0, 0)),
                pl.BlockSpec((2, _M, _HPAD), lambda i, *_: (0, 0, 0)),
                pl.BlockSpec((1, 2), lambda i, *_: (0, 0)),
            ),
        ),
        compiler_params=pltpu.CompilerParams(
            dimension_semantics=("arbitrary",),
            vmem_limit_bytes=32 * 1024 * 1024,
        ),
    )(c_disp, motion, *operands)


def kernel(gcn_output, motion, c_disp, h, c, node_positions,
           w_l0, b_l0, w_l1, b_l1, w_fc1, b_fc1, w_fc2, b_fc2, w_fc3, b_fc3):
    motion = motion.astype(jnp.int32)
    gr = gcn_output.reshape(3, 11, 3, 44)
    h0, c0, part = _run_lstm_pair(c_disp, motion, gr, h, c,
                                  b_l0, w_l0, w_l1)
    out, h_stack, c_stack, cdn = _run_head(
        c_disp, motion, part, b_l1, c, h0, c0,
        w_fc1, b_fc1, w_fc2, b_fc2, w_fc3, b_fc3)

    out = out[:, :_OUT].reshape(_EGO * _EGO, _NCLS)
    new_state = {
        "c_disp": cdn.reshape(2),
        "h": h_stack,
        "c": c_stack,
        "node_positions": node_positions,
    }
    return out, new_state


# speedup vs baseline: 1.0779x; 1.0779x over previous
import jax
import jax.numpy as jnp
from jax.experimental import pallas as pl
from jax.experimental.pallas import tpu as pltpu

_WIN = 11
_EGO = 33
_NCLS = 4
_IN_PAD = 768
_HPAD = 1024
_NH = 512
_M = 9
_OUT = _WIN * _WIN * _NCLS


def _iota_consts():
    a1 = jax.lax.broadcasted_iota(jnp.int32, (512, 1), 0)
    adiv = jnp.zeros_like(a1)
    for t in range(1, 12):
        adiv = adiv + (a1 >= 44 * t).astype(jnp.int32)
    amod = a1 - 44 * adiv
    tgt = (a1 % 4) * 121 + 11 * adiv + (amod // 4)
    tgt = jnp.where(a1 < 484, tgt, -1)
    b2 = jax.lax.broadcasted_iota(jnp.int32, (512, _IN_PAD), 1)
    p2 = (b2 == tgt).astype(jnp.bfloat16)

    p1 = jax.lax.broadcasted_iota(jnp.int32, (128, 1), 0)
    p1 = jnp.where(p1 < 121, p1, -1000)
    b3 = jax.lax.broadcasted_iota(jnp.int32, (128, _IN_PAD), 1)
    q4 = (b3 == 484 + p1).astype(jnp.bfloat16)
    q5 = (b3 == 605 + p1).astype(jnp.bfloat16)

    w1 = jax.lax.broadcasted_iota(jnp.int32, (11, 1), 0)
    c2 = jax.lax.broadcasted_iota(jnp.int32, (11, 512), 1)
    cdiv = jnp.zeros_like(c2)
    for t in range(1, 12):
        cdiv = cdiv + (c2 >= 44 * t).astype(jnp.int32)
    tm = ((cdiv == w1) & (c2 < 484)).astype(jnp.bfloat16)
    return p2, q4, q5, tm


def _lstm2_kernel(cd_ref, mo_ref, gr_ref, hp0_ref, hp1_ref, c0p_ref, b0_ref,
                  w0_hbm, w1_hbm,
                  h0_ref, c0_ref, part_ref,
                  w0_buf, w1a_buf, w1b_buf, sem0, sem1):
    n = pl.program_id(0)

    pltpu.make_async_copy(w0_hbm.at[:, n], w0_buf, sem0).start()
    pltpu.make_async_copy(w1_hbm.at[:, pl.ds(n * _NH, _NH), :],
                          w1a_buf, sem1.at[0]).start()
    pltpu.make_async_copy(w1_hbm.at[:, pl.ds(_HPAD + n * _NH, _NH), :],
                          w1b_buf, sem1.at[1]).start()

    p2, q4, q5, tm = _iota_consts()
    s0 = jnp.clip(34 + cd_ref[0] + mo_ref[0], 0, 67)
    s1 = jnp.clip(34 + cd_ref[1] + mo_ref[1], 0, 67)
    ones11 = jnp.ones((1, 11), jnp.bfloat16)
    rows = []
    for w in range(9):
        i, j = w // 3, w % 3
        awin = gr_ref[i, :, j, :].astype(jnp.bfloat16)
        atile = jnp.concatenate([awin] * 12, axis=1)[:, :512] * tm
        rows.append(jnp.dot(ones11, atile,
                            preferred_element_type=jnp.float32))
    vbig = jnp.concatenate(rows, axis=0).astype(jnp.bfloat16)

    r9 = jax.lax.broadcasted_iota(jnp.int32, (_M, 128), 0)
    c128 = jax.lax.broadcasted_iota(jnp.int32, (_M, 128), 1)
    ivec = (r9 >= 3).astype(jnp.int32) + (r9 >= 6).astype(jnp.int32)
    jvec = r9 - 3 * ivec
    wivec = jnp.zeros_like(c128)
    for t in range(1, 11):
        wivec = wivec + (c128 >= 11 * t).astype(jnp.int32)
    wjvec = c128 - 11 * wivec
    ch4 = (s0 + 11 * ivec + wivec).astype(jnp.float32) / 100.0
    ch5 = (s1 + 11 * jvec + wjvec).astype(jnp.float32) / 100.0

    d = (jnp.dot(vbig, p2, preferred_element_type=jnp.float32)
         + jnp.dot(ch4.astype(jnp.bfloat16), q4,
                   preferred_element_type=jnp.float32)
         + jnp.dot(ch5.astype(jnp.bfloat16), q5,
                   preferred_element_type=jnp.float32))
    db = d.astype(jnp.bfloat16)
    hp0b = hp0_ref[...].astype(jnp.bfloat16)

    pltpu.make_async_copy(w0_hbm.at[:, 0], w0_buf, sem0).wait()
    pre = []
    for g in range(4):
        b = b0_ref[0, pl.ds(g * 2 * _NH + n * _NH, _NH)]
        pre.append(b + jnp.dot(db, w0_buf[g][:_IN_PAD],
                               preferred_element_type=jnp.float32)
                   + jnp.dot(hp0b, w0_buf[g][_IN_PAD:],
                             preferred_element_type=jnp.float32))
    c0 = jax.nn.sigmoid(pre[1]) * c0p_ref[...] + \
        jax.nn.sigmoid(pre[0]) * jnp.tanh(pre[2])
    h0 = jax.nn.sigmoid(pre[3]) * jnp.tanh(c0)
    h0_ref[...] = h0
    c0_ref[...] = c0

    h0b = h0.astype(jnp.bfloat16)
    hpb = hp1_ref[...].astype(jnp.bfloat16)
    pltpu.make_async_copy(w1_hbm.at[:, pl.ds(0, _NH), :], w1a_buf,
                          sem1.at[0]).wait()
    pltpu.make_async_copy(w1_hbm.at[:, pl.ds(0, _NH), :], w1b_buf,
                          sem1.at[1]).wait()
    parts = []
    for blk in range(8):
        parts.append(
            jnp.dot(h0b, w1a_buf[blk], preferred_element_type=jnp.float32)
            + jnp.dot(hpb, w1b_buf[blk], preferred_element_type=jnp.float32))
    part_ref[0] = jnp.concatenate(parts, axis=1)


def _run_lstm_pair(c_disp, motion, gr, h0_prev, h1_prev, c0_prev, b0, w0, w1):
    w0r = w0.reshape(4, 2, _IN_PAD + _HPAD, _NH)
    return pl.pallas_call(
        _lstm2_kernel,
        out_shape=(
            jax.ShapeDtypeStruct((_M, _HPAD), jnp.float32),
            jax.ShapeDtypeStruct((_M, _HPAD), jnp.float32),
            jax.ShapeDtypeStruct((2, _M, 8 * _NH), jnp.float32),
        ),
        grid_spec=pltpu.PrefetchScalarGridSpec(
            num_scalar_prefetch=2,
            grid=(2,),
            in_specs=[
                pl.BlockSpec((3, 11, 3, 44), lambda n, *_: (0, 0, 0, 0)),
                pl.BlockSpec((_M, _HPAD), lambda n, *_: (0, 0)),
                pl.BlockSpec((_M, _NH), lambda n, *_: (0, n)),
                pl.BlockSpec((_M, _NH), lambda n, *_: (0, n)),
                pl.BlockSpec((1, 8 * _NH), lambda n, *_: (0, 0)),
                pl.BlockSpec(memory_space=pl.ANY),
                pl.BlockSpec(memory_space=pl.ANY),
            ],
            out_specs=(
                pl.BlockSpec((_M, _NH), lambda n, *_: (0, n)),
                pl.BlockSpec((_M, _NH), lambda n, *_: (0, n)),
                pl.BlockSpec((1, _M, 8 * _NH), lambda n, *_: (n, 0, 0)),
            ),
            scratch_shapes=[
                pltpu.VMEM((4, _IN_PAD + _HPAD, _NH), jnp.bfloat16),
                pltpu.VMEM((8, _NH, _NH), jnp.bfloat16),
                pltpu.VMEM((8, _NH, _NH), jnp.bfloat16),
                pltpu.SemaphoreType.DMA,
                pltpu.SemaphoreType.DMA((2,)),
            ],
        ),
        compiler_params=pltpu.CompilerParams(
            dimension_semantics=("parallel",),
            vmem_limit_bytes=50 * 1024 * 1024,
        ),
    )(c_disp, motion, gr, h0_prev, h1_prev, c0_prev, b0, w0r, w1)


def _head_kernel(cd_ref, mo_ref, p_ref, b1_ref, c1p_ref, h0_ref, c0_ref,
                 w1_ref, bf1_ref, w2_ref, bf2_ref, w3_ref, bf3_ref,
                 out_ref, hs_ref, cs_ref, cdn_ref):
    pre = p_ref[0] + p_ref[1] + b1_ref[...]
    gi = jax.nn.sigmoid(pre[:, 0 * _HPAD:1 * _HPAD])
    gf = jax.nn.sigmoid(pre[:, 1 * _HPAD:2 * _HPAD])
    gg = jnp.tanh(pre[:, 2 * _HPAD:3 * _HPAD])
    go = jax.nn.sigmoid(pre[:, 3 * _HPAD:4 * _HPAD])
    c1 = gf * c1p_ref[...] + gi * gg
    h1 = go * jnp.tanh(c1)
    hs_ref[0] = h0_ref[...]
    hs_ref[1] = h1
    cs_ref[0] = c0_ref[...]
    cs_ref[1] = c1
    lane = jax.lax.broadcasted_iota(jnp.int32, (1, 2), 1)
    cdn_ref[...] = jnp.where(lane == 0, cd_ref[0] + mo_ref[0],
                             cd_ref[1] + mo_ref[1])
    t = jnp.dot(h1.astype(jnp.bfloat16), w1_ref[...],
                preferred_element_type=jnp.float32) + bf1_ref[...]
    t = jnp.maximum(t, 0.0)
    t = jnp.dot(t.astype(jnp.bfloat16), w2_ref[...],
                preferred_element_type=jnp.float32) + bf2_ref[...]
    t = jnp.maximum(t, 0.0)
    out_ref[...] = jnp.dot(t.astype(jnp.bfloat16), w3_ref[...],
                           preferred_element_type=jnp.float32) + bf3_ref[...]


def _run_head(c_disp, motion, part, b1, c1_prev, h0, c0,
              w1, bf1, w2, bf2, w3, bf3):
    operands = (part, b1, c1_prev, h0, c0, w1, bf1, w2, bf2, w3, bf3)
    return pl.pallas_call(
        _head_kernel,
        out_shape=(
            jax.ShapeDtypeStruct((_M, 512), jnp.float32),
            jax.ShapeDtypeStruct((2, _M, _HPAD), jnp.float32),
            jax.ShapeDtypeStruct((2, _M, _HPAD), jnp.float32),
            jax.ShapeDtypeStruct((1, 2), jnp.int32),
        ),
        grid_spec=pltpu.PrefetchScalarGridSpec(
            num_scalar_prefetch=2,
            grid=(1,),
            in_specs=[pl.BlockSpec(op.shape, lambda i, *_, nd=op.ndim:
                                   (0,) * nd)
                      for op in operands],
            out_specs=(
                pl.BlockSpec((_M, 512), lambda i, *_: (0, 0)),
                pl.BlockSpec((2, _M, _HPAD), lambda i, *_: (0, 0, 0)),
                pl.BlockSpec((2, _M, _HPAD), lambda i, *_: (0, 0, 0)),
                pl.BlockSpec((1, 2), lambda i, *_: (0, 0)),
            ),
        ),
        compiler_params=pltpu.CompilerParams(
            dimension_semantics=("arbitrary",),
            vmem_limit_bytes=32 * 1024 * 1024,
        ),
    )(c_disp, motion, *operands)


def kernel(gcn_output, motion, c_disp, h, c, node_positions,
           w_l0, b_l0, w_l1, b_l1, w_fc1, b_fc1, w_fc2, b_fc2, w_fc3, b_fc3):
    motion = motion.astype(jnp.int32)
    gr = jnp.maximum(gcn_output, 0.0).reshape(3, 11, 3, 44)
    h0, c0, part = _run_lstm_pair(c_disp, motion, gr, h[0], h[1], c[0],
                                  b_l0, w_l0, w_l1)
    out, h_stack, c_stack, cdn = _run_head(
        c_disp, motion, part, b_l1, c[1], h0, c0,
        w_fc1, b_fc1, w_fc2, b_fc2, w_fc3, b_fc3)

    out = out[:, :_OUT].reshape(_EGO * _EGO, _NCLS)
    new_state = {
        "c_disp": cdn.reshape(2),
        "h": h_stack,
        "c": c_stack,
        "node_positions": node_positions,
    }
    return out, new_state


# speedup vs baseline: 1.1008x; 1.0213x over previous
import jax
import jax.numpy as jnp
from jax.experimental import pallas as pl
from jax.experimental.pallas import tpu as pltpu

_WIN = 11
_EGO = 33
_NCLS = 4
_IN_PAD = 768
_HPAD = 1024
_NH = 512
_M = 9
_OUT = _WIN * _WIN * _NCLS


def _iota_consts():
    a1 = jax.lax.broadcasted_iota(jnp.int32, (512, 1), 0)
    adiv = jnp.zeros_like(a1)
    for t in range(1, 12):
        adiv = adiv + (a1 >= 44 * t).astype(jnp.int32)
    amod = a1 - 44 * adiv
    tgt = (a1 % 4) * 121 + 11 * adiv + (amod // 4)
    tgt = jnp.where(a1 < 484, tgt, -1)
    b2 = jax.lax.broadcasted_iota(jnp.int32, (512, _IN_PAD), 1)
    p2 = (b2 == tgt).astype(jnp.bfloat16)

    p1 = jax.lax.broadcasted_iota(jnp.int32, (128, 1), 0)
    p1 = jnp.where(p1 < 121, p1, -1000)
    b3 = jax.lax.broadcasted_iota(jnp.int32, (128, _IN_PAD), 1)
    q4 = (b3 == 484 + p1).astype(jnp.bfloat16)
    q5 = (b3 == 605 + p1).astype(jnp.bfloat16)

    w1 = jax.lax.broadcasted_iota(jnp.int32, (11, 1), 0)
    c2 = jax.lax.broadcasted_iota(jnp.int32, (11, 512), 1)
    cdiv = jnp.zeros_like(c2)
    for t in range(1, 12):
        cdiv = cdiv + (c2 >= 44 * t).astype(jnp.int32)
    tm = ((cdiv == w1) & (c2 < 484)).astype(jnp.bfloat16)
    return p2, q4, q5, tm


def _lstm2_kernel(cd_ref, mo_ref, gr_ref, hp0_ref, hp1_ref, c0p_ref,
                  b0_hbm, w0_hbm, w1_hbm,
                  h0_ref, c0_ref, part_ref,
                  b0_buf, w0_buf, w1a_buf, w1b_buf, semb, sem0, sem1a, sem1b):
    n = pl.program_id(0)

    pltpu.make_async_copy(b0_hbm, b0_buf, semb).start()
    for g in range(4):
        pltpu.make_async_copy(w0_hbm.at[g, n], w0_buf.at[g], sem0).start()
    for blk in range(8):
        pltpu.make_async_copy(w1_hbm.at[blk, n], w1a_buf.at[blk],
                              sem1a).start()
        pltpu.make_async_copy(w1_hbm.at[blk, 2 + n], w1b_buf.at[blk],
                              sem1b).start()

    p2, q4, q5, tm = _iota_consts()
    s0 = jnp.clip(34 + cd_ref[0] + mo_ref[0], 0, 67)
    s1 = jnp.clip(34 + cd_ref[1] + mo_ref[1], 0, 67)
    ones11 = jnp.ones((1, 11), jnp.bfloat16)
    rows = []
    for w in range(9):
        i, j = w // 3, w % 3
        awin = gr_ref[i, :, j, :].astype(jnp.bfloat16)
        atile = jnp.concatenate([awin] * 12, axis=1)[:, :512] * tm
        rows.append(jnp.dot(ones11, atile,
                            preferred_element_type=jnp.float32))
    vbig = jnp.concatenate(rows, axis=0).astype(jnp.bfloat16)

    r9 = jax.lax.broadcasted_iota(jnp.int32, (_M, 128), 0)
    c128 = jax.lax.broadcasted_iota(jnp.int32, (_M, 128), 1)
    ivec = (r9 >= 3).astype(jnp.int32) + (r9 >= 6).astype(jnp.int32)
    jvec = r9 - 3 * ivec
    wivec = jnp.zeros_like(c128)
    for t in range(1, 11):
        wivec = wivec + (c128 >= 11 * t).astype(jnp.int32)
    wjvec = c128 - 11 * wivec
    ch4 = (s0 + 11 * ivec + wivec).astype(jnp.float32) / 100.0
    ch5 = (s1 + 11 * jvec + wjvec).astype(jnp.float32) / 100.0

    d = (jnp.dot(vbig, p2, preferred_element_type=jnp.float32)
         + jnp.dot(ch4.astype(jnp.bfloat16), q4,
                   preferred_element_type=jnp.float32)
         + jnp.dot(ch5.astype(jnp.bfloat16), q5,
                   preferred_element_type=jnp.float32))
    db = d.astype(jnp.bfloat16)
    hp0b = hp0_ref[0].astype(jnp.bfloat16)

    pltpu.make_async_copy(w0_hbm.at[:, 0], w0_buf, sem0).wait()
    pltpu.make_async_copy(b0_hbm, b0_buf, semb).wait()
    pre = []
    for g in range(4):
        b = b0_buf[0, pl.ds(g * 2 * _NH + n * _NH, _NH)]
        pre.append(b + jnp.dot(db, w0_buf[g][:_IN_PAD],
                               preferred_element_type=jnp.float32)
                   + jnp.dot(hp0b, w0_buf[g][_IN_PAD:],
                             preferred_element_type=jnp.float32))
    c0 = jax.nn.sigmoid(pre[1]) * c0p_ref[0] + \
        jax.nn.sigmoid(pre[0]) * jnp.tanh(pre[2])
    h0 = jax.nn.sigmoid(pre[3]) * jnp.tanh(c0)
    h0_ref[...] = h0
    c0_ref[...] = c0

    h0b = h0.astype(jnp.bfloat16)
    hpb = hp1_ref[0].astype(jnp.bfloat16)
    pltpu.make_async_copy(w1_hbm.at[:, 0], w1a_buf, sem1a).wait()
    pltpu.make_async_copy(w1_hbm.at[:, 0], w1b_buf, sem1b).wait()
    parts = []
    for blk in range(8):
        parts.append(
            jnp.dot(h0b, w1a_buf[blk], preferred_element_type=jnp.float32)
            + jnp.dot(hpb, w1b_buf[blk], preferred_element_type=jnp.float32))
    part_ref[0] = jnp.concatenate(parts, axis=1)


def _run_lstm_pair(c_disp, motion, gr, h_all, c_all, b0, w0, w1):
    w0r = w0.reshape(4, 2, _IN_PAD + _HPAD, _NH)
    w1r = w1.reshape(8, 4, _NH, _NH)
    return pl.pallas_call(
        _lstm2_kernel,
        out_shape=(
            jax.ShapeDtypeStruct((_M, _HPAD), jnp.float32),
            jax.ShapeDtypeStruct((_M, _HPAD), jnp.float32),
            jax.ShapeDtypeStruct((2, _M, 8 * _NH), jnp.float32),
        ),
        grid_spec=pltpu.PrefetchScalarGridSpec(
            num_scalar_prefetch=2,
            grid=(2,),
            in_specs=[
                pl.BlockSpec((3, 11, 3, 44), lambda n, *_: (0, 0, 0, 0)),
                pl.BlockSpec((1, _M, _HPAD), lambda n, *_: (0, 0, 0)),
                pl.BlockSpec((1, _M, _NH), lambda n, *_: (1, 0, n)),
                pl.BlockSpec((1, _M, _NH), lambda n, *_: (0, 0, n)),
                pl.BlockSpec(memory_space=pl.ANY),
                pl.BlockSpec(memory_space=pl.ANY),
                pl.BlockSpec(memory_space=pl.ANY),
            ],
            out_specs=(
                pl.BlockSpec((_M, _NH), lambda n, *_: (0, n)),
                pl.BlockSpec((_M, _NH), lambda n, *_: (0, n)),
                pl.BlockSpec((1, _M, 8 * _NH), lambda n, *_: (n, 0, 0)),
            ),
            scratch_shapes=[
                pltpu.VMEM((1, 8 * _NH), jnp.float32),
                pltpu.VMEM((4, _IN_PAD + _HPAD, _NH), jnp.bfloat16),
                pltpu.VMEM((8, _NH, _NH), jnp.bfloat16),
                pltpu.VMEM((8, _NH, _NH), jnp.bfloat16),
                pltpu.SemaphoreType.DMA,
                pltpu.SemaphoreType.DMA,
                pltpu.SemaphoreType.DMA,
                pltpu.SemaphoreType.DMA,
            ],
        ),
        compiler_params=pltpu.CompilerParams(
            dimension_semantics=("parallel",),
            vmem_limit_bytes=50 * 1024 * 1024,
        ),
    )(c_disp, motion, gr, h_all, h_all, c_all, b0, w0r, w1r)


def _head_kernel(cd_ref, mo_ref, p_ref, b1_ref, c1p_ref, h0_ref, c0_ref,
                 w1_ref, bf1_ref, w2_ref, bf2_ref, w3_ref, bf3_ref,
                 out_ref, hs_ref, cs_ref, cdn_ref):
    pre = p_ref[0] + p_ref[1] + b1_ref[...]
    gi = jax.nn.sigmoid(pre[:, 0 * _HPAD:1 * _HPAD])
    gf = jax.nn.sigmoid(pre[:, 1 * _HPAD:2 * _HPAD])
    gg = jnp.tanh(pre[:, 2 * _HPAD:3 * _HPAD])
    go = jax.nn.sigmoid(pre[:, 3 * _HPAD:4 * _HPAD])
    c1 = gf * c1p_ref[0] + gi * gg
    h1 = go * jnp.tanh(c1)
    hs_ref[0] = h0_ref[...]
    hs_ref[1] = h1
    cs_ref[0] = c0_ref[...]
    cs_ref[1] = c1
    lane = jax.lax.broadcasted_iota(jnp.int32, (1, 2), 1)
    cdn_ref[...] = jnp.where(lane == 0, cd_ref[0] + mo_ref[0],
                             cd_ref[1] + mo_ref[1])
    t = jnp.dot(h1.astype(jnp.bfloat16), w1_ref[...],
                preferred_element_type=jnp.float32) + bf1_ref[...]
    t = jnp.maximum(t, 0.0)
    t = jnp.dot(t.astype(jnp.bfloat16), w2_ref[...],
                preferred_element_type=jnp.float32) + bf2_ref[...]
    t = jnp.maximum(t, 0.0)
    out_ref[...] = jnp.dot(t.astype(jnp.bfloat16), w3_ref[...],
                           preferred_element_type=jnp.float32) + bf3_ref[...]


def _run_head(c_disp, motion, part, b1, c1_prev, h0, c0,
              w1, bf1, w2, bf2, w3, bf3):
    operands = (part, b1, c1_prev, h0, c0, w1, bf1, w2, bf2, w3, bf3)
    in_specs = [pl.BlockSpec(op.shape, lambda i, *_, nd=op.ndim: (0,) * nd)
                for op in operands]
    in_specs[2] = pl.BlockSpec((1, _M, _HPAD), lambda i, *_: (1, 0, 0))
    return pl.pallas_call(
        _head_kernel,
        out_shape=(
            jax.ShapeDtypeStruct((_M, 512), jnp.float32),
            jax.ShapeDtypeStruct((2, _M, _HPAD), jnp.float32),
            jax.ShapeDtypeStruct((2, _M, _HPAD), jnp.float32),
            jax.ShapeDtypeStruct((1, 2), jnp.int32),
        ),
        grid_spec=pltpu.PrefetchScalarGridSpec(
            num_scalar_prefetch=2,
            grid=(1,),
            in_specs=in_specs,
            out_specs=(
                pl.BlockSpec((_M, 512), lambda i, *_: (0, 0)),
                pl.BlockSpec((2, _M, _HPAD), lambda i, *_: (0, 0, 0)),
                pl.BlockSpec((2, _M, _HPAD), lambda i, *_: (0, 0, 0)),
                pl.BlockSpec((1, 2), lambda i, *_: (0, 0)),
            ),
        ),
        compiler_params=pltpu.CompilerParams(
            dimension_semantics=("arbitrary",),
            vmem_limit_bytes=32 * 1024 * 1024,
        ),
    )(c_disp, motion, *operands)


def kernel(gcn_output, motion, c_disp, h, c, node_positions,
           w_l0, b_l0, w_l1, b_l1, w_fc1, b_fc1, w_fc2, b_fc2, w_fc3, b_fc3):
    motion = motion.astype(jnp.int32)
    gr = jnp.maximum(gcn_output, 0.0).reshape(3, 11, 3, 44)
    h0, c0, part = _run_lstm_pair(c_disp, motion, gr, h, c,
                                  b_l0, w_l0, w_l1)
    out, h_stack, c_stack, cdn = _run_head(
        c_disp, motion, part, b_l1, c, h0, c0,
        w_fc1, b_fc1, w_fc2, b_fc2, w_fc3, b_fc3)

    out = out[:, :_OUT].reshape(_EGO * _EGO, _NCLS)
    new_state = {
        "c_disp": cdn.reshape(2),
        "h": h_stack,
        "c": c_stack,
        "node_positions": node_positions,
    }
    return out, new_state


# speedup vs baseline: 1.2093x; 1.0985x over previous
import jax
import jax.numpy as jnp
from jax.experimental import pallas as pl
from jax.experimental.pallas import tpu as pltpu

_WIN = 11
_EGO = 33
_NCLS = 4
_IN_PAD = 768
_HPAD = 1024
_NH = 512
_M = 9
_OUT = _WIN * _WIN * _NCLS


def _iota_consts():
    a1 = jax.lax.broadcasted_iota(jnp.int32, (512, 1), 0)
    adiv = jnp.zeros_like(a1)
    for t in range(1, 12):
        adiv = adiv + (a1 >= 44 * t).astype(jnp.int32)
    amod = a1 - 44 * adiv
    tgt = (a1 % 4) * 121 + 11 * adiv + (amod // 4)
    tgt = jnp.where(a1 < 484, tgt, -1)
    b2 = jax.lax.broadcasted_iota(jnp.int32, (512, _IN_PAD), 1)
    p2 = (b2 == tgt).astype(jnp.bfloat16)

    p1 = jax.lax.broadcasted_iota(jnp.int32, (128, 1), 0)
    p1 = jnp.where(p1 < 121, p1, -1000)
    b3 = jax.lax.broadcasted_iota(jnp.int32, (128, _IN_PAD), 1)
    q4 = (b3 == 484 + p1).astype(jnp.bfloat16)
    q5 = (b3 == 605 + p1).astype(jnp.bfloat16)

    w1 = jax.lax.broadcasted_iota(jnp.int32, (11, 1), 0)
    c2 = jax.lax.broadcasted_iota(jnp.int32, (11, 512), 1)
    cdiv = jnp.zeros_like(c2)
    for t in range(1, 12):
        cdiv = cdiv + (c2 >= 44 * t).astype(jnp.int32)
    tm = ((cdiv == w1) & (c2 < 484)).astype(jnp.bfloat16)
    return p2, q4, q5, tm


def _lstm2_kernel(cd_ref, mo_ref, gr_ref, hp0_ref, hp1_ref, c0p_ref,
                  b0_hbm, w0_hbm, w1_hbm,
                  h0_ref, c0_ref, part_ref,
                  b0_buf, w0_buf, w1a_buf, w1b_buf, semb, sem0, sem1a, sem1b):
    n = pl.program_id(0)

    pltpu.make_async_copy(b0_hbm, b0_buf, semb).start()
    for g in range(4):
        pltpu.make_async_copy(w0_hbm.at[g, n], w0_buf.at[g], sem0).start()
    for k in range(4):
        for blk in (2 * k, 2 * k + 1):
            pltpu.make_async_copy(w1_hbm.at[blk, n], w1a_buf.at[blk],
                                  sem1a.at[k]).start(priority=1)
            pltpu.make_async_copy(w1_hbm.at[blk, 2 + n], w1b_buf.at[blk],
                                  sem1b.at[k]).start(priority=1)

    p2, q4, q5, tm = _iota_consts()
    s0 = jnp.clip(34 + cd_ref[0] + mo_ref[0], 0, 67)
    s1 = jnp.clip(34 + cd_ref[1] + mo_ref[1], 0, 67)
    ones11 = jnp.ones((1, 11), jnp.bfloat16)
    rows = []
    for w in range(9):
        i, j = w // 3, w % 3
        awin = gr_ref[i, :, j, :].astype(jnp.bfloat16)
        atile = jnp.concatenate([awin] * 12, axis=1)[:, :512] * tm
        rows.append(jnp.dot(ones11, atile,
                            preferred_element_type=jnp.float32))
    vbig = jnp.concatenate(rows, axis=0).astype(jnp.bfloat16)

    r9 = jax.lax.broadcasted_iota(jnp.int32, (_M, 128), 0)
    c128 = jax.lax.broadcasted_iota(jnp.int32, (_M, 128), 1)
    ivec = (r9 >= 3).astype(jnp.int32) + (r9 >= 6).astype(jnp.int32)
    jvec = r9 - 3 * ivec
    wivec = jnp.zeros_like(c128)
    for t in range(1, 11):
        wivec = wivec + (c128 >= 11 * t).astype(jnp.int32)
    wjvec = c128 - 11 * wivec
    ch4 = (s0 + 11 * ivec + wivec).astype(jnp.float32) / 100.0
    ch5 = (s1 + 11 * jvec + wjvec).astype(jnp.float32) / 100.0

    d = (jnp.dot(vbig, p2, preferred_element_type=jnp.float32)
         + jnp.dot(ch4.astype(jnp.bfloat16), q4,
                   preferred_element_type=jnp.float32)
         + jnp.dot(ch5.astype(jnp.bfloat16), q5,
                   preferred_element_type=jnp.float32))
    db = d.astype(jnp.bfloat16)
    hp0b = hp0_ref[0].astype(jnp.bfloat16)

    pltpu.make_async_copy(w0_hbm.at[:, 0], w0_buf, sem0).wait()
    pltpu.make_async_copy(b0_hbm, b0_buf, semb).wait()
    pre = []
    for g in range(4):
        b = b0_buf[0, pl.ds(g * 2 * _NH + n * _NH, _NH)]
        pre.append(b + jnp.dot(db, w0_buf[g][:_IN_PAD],
                               preferred_element_type=jnp.float32)
                   + jnp.dot(hp0b, w0_buf[g][_IN_PAD:],
                             preferred_element_type=jnp.float32))
    c0 = jax.nn.sigmoid(pre[1]) * c0p_ref[0] + \
        jax.nn.sigmoid(pre[0]) * jnp.tanh(pre[2])
    h0 = jax.nn.sigmoid(pre[3]) * jnp.tanh(c0)
    h0_ref[...] = h0
    c0_ref[...] = c0

    h0b = h0.astype(jnp.bfloat16)
    hpb = hp1_ref[0].astype(jnp.bfloat16)
    parts = []
    for k in range(4):
        pltpu.make_async_copy(w1_hbm.at[pl.ds(0, 2), 0],
                              w1a_buf.at[pl.ds(2 * k, 2)], sem1a.at[k]).wait()
        pltpu.make_async_copy(w1_hbm.at[pl.ds(0, 2), 0],
                              w1b_buf.at[pl.ds(2 * k, 2)], sem1b.at[k]).wait()
        for blk in (2 * k, 2 * k + 1):
            parts.append(
                jnp.dot(h0b, w1a_buf[blk],
                        preferred_element_type=jnp.float32)
                + jnp.dot(hpb, w1b_buf[blk],
                          preferred_element_type=jnp.float32))
    part_ref[0] = jnp.concatenate(parts, axis=1)


def _run_lstm_pair(c_disp, motion, gr, h_all, c_all, b0, w0, w1):
    w0r = w0.reshape(4, 2, _IN_PAD + _HPAD, _NH)
    w1r = w1.reshape(8, 4, _NH, _NH)
    return pl.pallas_call(
        _lstm2_kernel,
        out_shape=(
            jax.ShapeDtypeStruct((_M, _HPAD), jnp.float32),
            jax.ShapeDtypeStruct((_M, _HPAD), jnp.float32),
            jax.ShapeDtypeStruct((2, _M, 8 * _NH), jnp.float32),
        ),
        grid_spec=pltpu.PrefetchScalarGridSpec(
            num_scalar_prefetch=2,
            grid=(2,),
            in_specs=[
                pl.BlockSpec((3, 11, 3, 44), lambda n, *_: (0, 0, 0, 0)),
                pl.BlockSpec((1, _M, _HPAD), lambda n, *_: (0, 0, 0)),
                pl.BlockSpec((1, _M, _NH), lambda n, *_: (1, 0, n)),
                pl.BlockSpec((1, _M, _NH), lambda n, *_: (0, 0, n)),
                pl.BlockSpec(memory_space=pl.ANY),
                pl.BlockSpec(memory_space=pl.ANY),
                pl.BlockSpec(memory_space=pl.ANY),
            ],
            out_specs=(
                pl.BlockSpec((_M, _NH), lambda n, *_: (0, n)),
                pl.BlockSpec((_M, _NH), lambda n, *_: (0, n)),
                pl.BlockSpec((1, _M, 8 * _NH), lambda n, *_: (n, 0, 0)),
            ),
            scratch_shapes=[
                pltpu.VMEM((1, 8 * _NH), jnp.float32),
                pltpu.VMEM((4, _IN_PAD + _HPAD, _NH), jnp.bfloat16),
                pltpu.VMEM((8, _NH, _NH), jnp.bfloat16),
                pltpu.VMEM((8, _NH, _NH), jnp.bfloat16),
                pltpu.SemaphoreType.DMA,
                pltpu.SemaphoreType.DMA,
                pltpu.SemaphoreType.DMA((4,)),
                pltpu.SemaphoreType.DMA((4,)),
            ],
        ),
        compiler_params=pltpu.CompilerParams(
            dimension_semantics=("parallel",),
            vmem_limit_bytes=50 * 1024 * 1024,
        ),
    )(c_disp, motion, gr, h_all, h_all, c_all, b0, w0r, w1r)


def _head_kernel(cd_ref, mo_ref, p_ref, b1_ref, c1p_ref, h0_ref, c0_ref,
                 w1_ref, bf1_ref, w2_ref, bf2_ref, w3_ref, bf3_ref,
                 out_ref, hs_ref, cs_ref, cdn_ref):
    pre = p_ref[0] + p_ref[1] + b1_ref[...]
    gi = jax.nn.sigmoid(pre[:, 0 * _HPAD:1 * _HPAD])
    gf = jax.nn.sigmoid(pre[:, 1 * _HPAD:2 * _HPAD])
    gg = jnp.tanh(pre[:, 2 * _HPAD:3 * _HPAD])
    go = jax.nn.sigmoid(pre[:, 3 * _HPAD:4 * _HPAD])
    c1 = gf * c1p_ref[0] + gi * gg
    h1 = go * jnp.tanh(c1)
    hs_ref[0] = h0_ref[...]
    hs_ref[1] = h1
    cs_ref[0] = c0_ref[...]
    cs_ref[1] = c1
    lane = jax.lax.broadcasted_iota(jnp.int32, (1, 2), 1)
    cdn_ref[...] = jnp.where(lane == 0, cd_ref[0] + mo_ref[0],
                             cd_ref[1] + mo_ref[1])
    t = jnp.dot(h1.astype(jnp.bfloat16), w1_ref[...],
                preferred_element_type=jnp.float32) + bf1_ref[...]
    t = jnp.maximum(t, 0.0)
    t = jnp.dot(t.astype(jnp.bfloat16), w2_ref[...],
                preferred_element_type=jnp.float32) + bf2_ref[...]
    t = jnp.maximum(t, 0.0)
    out_ref[...] = jnp.dot(t.astype(jnp.bfloat16), w3_ref[...],
                           preferred_element_type=jnp.float32) + bf3_ref[...]


def _run_head(c_disp, motion, part, b1, c1_prev, h0, c0,
              w1, bf1, w2, bf2, w3, bf3):
    operands = (part, b1, c1_prev, h0, c0, w1, bf1, w2, bf2, w3, bf3)
    in_specs = [pl.BlockSpec(op.shape, lambda i, *_, nd=op.ndim: (0,) * nd)
                for op in operands]
    in_specs[2] = pl.BlockSpec((1, _M, _HPAD), lambda i, *_: (1, 0, 0))
    return pl.pallas_call(
        _head_kernel,
        out_shape=(
            jax.ShapeDtypeStruct((_M, 512), jnp.float32),
            jax.ShapeDtypeStruct((2, _M, _HPAD), jnp.float32),
            jax.ShapeDtypeStruct((2, _M, _HPAD), jnp.float32),
            jax.ShapeDtypeStruct((1, 2), jnp.int32),
        ),
        grid_spec=pltpu.PrefetchScalarGridSpec(
            num_scalar_prefetch=2,
            grid=(1,),
            in_specs=in_specs,
            out_specs=(
                pl.BlockSpec((_M, 512), lambda i, *_: (0, 0)),
                pl.BlockSpec((2, _M, _HPAD), lambda i, *_: (0, 0, 0)),
                pl.BlockSpec((2, _M, _HPAD), lambda i, *_: (0, 0, 0)),
                pl.BlockSpec((1, 2), lambda i, *_: (0, 0)),
            ),
        ),
        compiler_params=pltpu.CompilerParams(
            dimension_semantics=("arbitrary",),
            vmem_limit_bytes=32 * 1024 * 1024,
        ),
    )(c_disp, motion, *operands)


def kernel(gcn_output, motion, c_disp, h, c, node_positions,
           w_l0, b_l0, w_l1, b_l1, w_fc1, b_fc1, w_fc2, b_fc2, w_fc3, b_fc3):
    motion = motion.astype(jnp.int32)
    gr = jnp.maximum(gcn_output, 0.0).reshape(3, 11, 3, 44)
    h0, c0, part = _run_lstm_pair(c_disp, motion, gr, h, c,
                                  b_l0, w_l0, w_l1)
    out, h_stack, c_stack, cdn = _run_head(
        c_disp, motion, part, b_l1, c, h0, c0,
        w_fc1, b_fc1, w_fc2, b_fc2, w_fc3, b_fc3)

    out = out[:, :_OUT].reshape(_EGO * _EGO, _NCLS)
    new_state = {
        "c_disp": cdn.reshape(2),
        "h": h_stack,
        "c": c_stack,
        "node_positions": node_positions,
    }
    return out, new_state


# speedup vs baseline: 1.2575x; 1.0398x over previous
import jax
import jax.numpy as jnp
from jax.experimental import pallas as pl
from jax.experimental.pallas import tpu as pltpu

_WIN = 11
_EGO = 33
_NCLS = 4
_IN_PAD = 768
_HPAD = 1024
_NH = 512
_M = 9
_OUT = _WIN * _WIN * _NCLS


def _iota_consts():
    a1 = jax.lax.broadcasted_iota(jnp.int32, (512, 1), 0)
    adiv = jnp.zeros_like(a1)
    for t in range(1, 12):
        adiv = adiv + (a1 >= 44 * t).astype(jnp.int32)
    amod = a1 - 44 * adiv
    tgt = (a1 % 4) * 121 + 11 * adiv + (amod // 4)
    tgt = jnp.where(a1 < 484, tgt, -1)
    b2 = jax.lax.broadcasted_iota(jnp.int32, (512, _IN_PAD), 1)
    p2 = (b2 == tgt).astype(jnp.bfloat16)

    p1 = jax.lax.broadcasted_iota(jnp.int32, (128, 1), 0)
    p1 = jnp.where(p1 < 121, p1, -1000)
    b3 = jax.lax.broadcasted_iota(jnp.int32, (128, _IN_PAD), 1)
    q4 = (b3 == 484 + p1).astype(jnp.bfloat16)
    q5 = (b3 == 605 + p1).astype(jnp.bfloat16)

    w1 = jax.lax.broadcasted_iota(jnp.int32, (11, 1), 0)
    c2 = jax.lax.broadcasted_iota(jnp.int32, (11, 512), 1)
    cdiv = jnp.zeros_like(c2)
    for t in range(1, 12):
        cdiv = cdiv + (c2 >= 44 * t).astype(jnp.int32)
    tm = ((cdiv == w1) & (c2 < 484)).astype(jnp.bfloat16)
    return p2, q4, q5, tm


def _lstm2_kernel(cd_ref, mo_ref, gr_ref, hp0_ref, hp1_ref, c0p_ref,
                  b0_hbm, w0_hbm, w1_hbm,
                  h0_ref, c0_ref, part_ref,
                  b0_buf, w0_buf, w1a_buf, w1b_buf, semb, sem0, sem1a, sem1b):
    n = pl.program_id(0)

    pltpu.make_async_copy(b0_hbm, b0_buf, semb).start()
    for g in range(4):
        pltpu.make_async_copy(w0_hbm.at[g, n], w0_buf.at[g],
                              sem0.at[g]).start()
    for k in range(4):
        for blk in (2 * k, 2 * k + 1):
            pltpu.make_async_copy(w1_hbm.at[blk, n], w1a_buf.at[blk],
                                  sem1a.at[k]).start(priority=1)
            pltpu.make_async_copy(w1_hbm.at[blk, 2 + n], w1b_buf.at[blk],
                                  sem1b.at[k]).start(priority=1)

    p2, q4, q5, tm = _iota_consts()
    s0 = jnp.clip(34 + cd_ref[0] + mo_ref[0], 0, 67)
    s1 = jnp.clip(34 + cd_ref[1] + mo_ref[1], 0, 67)
    ones11 = jnp.ones((1, 11), jnp.bfloat16)
    rows = []
    for w in range(9):
        i, j = w // 3, w % 3
        awin = gr_ref[i, :, j, :].astype(jnp.bfloat16)
        atile = jnp.concatenate([awin] * 12, axis=1)[:, :512] * tm
        rows.append(jnp.dot(ones11, atile,
                            preferred_element_type=jnp.float32))
    vbig = jnp.concatenate(rows, axis=0).astype(jnp.bfloat16)

    r9 = jax.lax.broadcasted_iota(jnp.int32, (_M, 128), 0)
    c128 = jax.lax.broadcasted_iota(jnp.int32, (_M, 128), 1)
    ivec = (r9 >= 3).astype(jnp.int32) + (r9 >= 6).astype(jnp.int32)
    jvec = r9 - 3 * ivec
    wivec = jnp.zeros_like(c128)
    for t in range(1, 11):
        wivec = wivec + (c128 >= 11 * t).astype(jnp.int32)
    wjvec = c128 - 11 * wivec
    ch4 = (s0 + 11 * ivec + wivec).astype(jnp.float32) / 100.0
    ch5 = (s1 + 11 * jvec + wjvec).astype(jnp.float32) / 100.0

    d = (jnp.dot(vbig, p2, preferred_element_type=jnp.float32)
         + jnp.dot(ch4.astype(jnp.bfloat16), q4,
                   preferred_element_type=jnp.float32)
         + jnp.dot(ch5.astype(jnp.bfloat16), q5,
                   preferred_element_type=jnp.float32))
    db = d.astype(jnp.bfloat16)
    hp0b = hp0_ref[0].astype(jnp.bfloat16)

    pltpu.make_async_copy(b0_hbm, b0_buf, semb).wait()
    pre = []
    for g in range(4):
        pltpu.make_async_copy(w0_hbm.at[0, 0], w0_buf.at[g],
                              sem0.at[g]).wait()
        b = b0_buf[0, pl.ds(g * 2 * _NH + n * _NH, _NH)]
        pre.append(b + jnp.dot(db, w0_buf[g][:_IN_PAD],
                               preferred_element_type=jnp.float32)
                   + jnp.dot(hp0b, w0_buf[g][_IN_PAD:],
                             preferred_element_type=jnp.float32))
    c0 = jax.nn.sigmoid(pre[1]) * c0p_ref[0] + \
        jax.nn.sigmoid(pre[0]) * jnp.tanh(pre[2])
    h0 = jax.nn.sigmoid(pre[3]) * jnp.tanh(c0)
    h0_ref[...] = h0
    c0_ref[...] = c0

    h0b = h0.astype(jnp.bfloat16)
    hpb = hp1_ref[0].astype(jnp.bfloat16)
    parts = []
    for k in range(4):
        pltpu.make_async_copy(w1_hbm.at[pl.ds(0, 2), 0],
                              w1a_buf.at[pl.ds(2 * k, 2)], sem1a.at[k]).wait()
        pltpu.make_async_copy(w1_hbm.at[pl.ds(0, 2), 0],
                              w1b_buf.at[pl.ds(2 * k, 2)], sem1b.at[k]).wait()
        for blk in (2 * k, 2 * k + 1):
            parts.append(
                jnp.dot(h0b, w1a_buf[blk],
                        preferred_element_type=jnp.float32)
                + jnp.dot(hpb, w1b_buf[blk],
                          preferred_element_type=jnp.float32))
    part_ref[0] = jnp.concatenate(parts, axis=1)


def _run_lstm_pair(c_disp, motion, gr, h_all, c_all, b0, w0, w1):
    w0r = w0.reshape(4, 2, _IN_PAD + _HPAD, _NH)
    w1r = w1.reshape(8, 4, _NH, _NH)
    return pl.pallas_call(
        _lstm2_kernel,
        out_shape=(
            jax.ShapeDtypeStruct((_M, _HPAD), jnp.float32),
            jax.ShapeDtypeStruct((_M, _HPAD), jnp.float32),
            jax.ShapeDtypeStruct((2, _M, 8 * _NH), jnp.float32),
        ),
        grid_spec=pltpu.PrefetchScalarGridSpec(
            num_scalar_prefetch=2,
            grid=(2,),
            in_specs=[
                pl.BlockSpec((3, 11, 3, 44), lambda n, *_: (0, 0, 0, 0)),
                pl.BlockSpec((1, _M, _HPAD), lambda n, *_: (0, 0, 0)),
                pl.BlockSpec((1, _M, _NH), lambda n, *_: (1, 0, n)),
                pl.BlockSpec((1, _M, _NH), lambda n, *_: (0, 0, n)),
                pl.BlockSpec(memory_space=pl.ANY),
                pl.BlockSpec(memory_space=pl.ANY),
                pl.BlockSpec(memory_space=pl.ANY),
            ],
            out_specs=(
                pl.BlockSpec((_M, _NH), lambda n, *_: (0, n)),
                pl.BlockSpec((_M, _NH), lambda n, *_: (0, n)),
                pl.BlockSpec((1, _M, 8 * _NH), lambda n, *_: (n, 0, 0)),
            ),
            scratch_shapes=[
                pltpu.VMEM((1, 8 * _NH), jnp.float32),
                pltpu.VMEM((4, _IN_PAD + _HPAD, _NH), jnp.bfloat16),
                pltpu.VMEM((8, _NH, _NH), jnp.bfloat16),
                pltpu.VMEM((8, _NH, _NH), jnp.bfloat16),
                pltpu.SemaphoreType.DMA,
                pltpu.SemaphoreType.DMA((4,)),
                pltpu.SemaphoreType.DMA((4,)),
                pltpu.SemaphoreType.DMA((4,)),
            ],
        ),
        compiler_params=pltpu.CompilerParams(
            dimension_semantics=("parallel",),
            vmem_limit_bytes=50 * 1024 * 1024,
        ),
    )(c_disp, motion, gr, h_all, h_all, c_all, b0, w0r, w1r)


def _head_kernel(cd_ref, mo_ref, p_ref, b1_ref, c1p_ref, h0_ref, c0_ref,
                 w1_ref, bf1_ref, w2_ref, bf2_ref, w3_ref, bf3_ref,
                 out_ref, hs_ref, cs_ref, cdn_ref):
    pre = p_ref[0] + p_ref[1] + b1_ref[...]
    gi = jax.nn.sigmoid(pre[:, 0 * _HPAD:1 * _HPAD])
    gf = jax.nn.sigmoid(pre[:, 1 * _HPAD:2 * _HPAD])
    gg = jnp.tanh(pre[:, 2 * _HPAD:3 * _HPAD])
    go = jax.nn.sigmoid(pre[:, 3 * _HPAD:4 * _HPAD])
    c1 = gf * c1p_ref[0] + gi * gg
    h1 = go * jnp.tanh(c1)
    hs_ref[0] = h0_ref[...]
    hs_ref[1] = h1
    cs_ref[0] = c0_ref[...]
    cs_ref[1] = c1
    lane = jax.lax.broadcasted_iota(jnp.int32, (1, 2), 1)
    cdn_ref[...] = jnp.where(lane == 0, cd_ref[0] + mo_ref[0],
                             cd_ref[1] + mo_ref[1])
    t = jnp.dot(h1.astype(jnp.bfloat16), w1_ref[...],
                preferred_element_type=jnp.float32) + bf1_ref[...]
    t = jnp.maximum(t, 0.0)
    t = jnp.dot(t.astype(jnp.bfloat16), w2_ref[...],
                preferred_element_type=jnp.float32) + bf2_ref[...]
    t = jnp.maximum(t, 0.0)
    out_ref[...] = jnp.dot(t.astype(jnp.bfloat16), w3_ref[...],
                           preferred_element_type=jnp.float32) + bf3_ref[...]


def _run_head(c_disp, motion, part, b1, c1_prev, h0, c0,
              w1, bf1, w2, bf2, w3, bf3):
    operands = (part, b1, c1_prev, h0, c0, w1, bf1, w2, bf2, w3, bf3)
    in_specs = [pl.BlockSpec(op.shape, lambda i, *_, nd=op.ndim: (0,) * nd)
                for op in operands]
    in_specs[2] = pl.BlockSpec((1, _M, _HPAD), lambda i, *_: (1, 0, 0))
    return pl.pallas_call(
        _head_kernel,
        out_shape=(
            jax.ShapeDtypeStruct((_M, 512), jnp.float32),
            jax.ShapeDtypeStruct((2, _M, _HPAD), jnp.float32),
            jax.ShapeDtypeStruct((2, _M, _HPAD), jnp.float32),
            jax.ShapeDtypeStruct((1, 2), jnp.int32),
        ),
        grid_spec=pltpu.PrefetchScalarGridSpec(
            num_scalar_prefetch=2,
            grid=(1,),
            in_specs=in_specs,
            out_specs=(
                pl.BlockSpec((_M, 512), lambda i, *_: (0, 0)),
                pl.BlockSpec((2, _M, _HPAD), lambda i, *_: (0, 0, 0)),
                pl.BlockSpec((2, _M, _HPAD), lambda i, *_: (0, 0, 0)),
                pl.BlockSpec((1, 2), lambda i, *_: (0, 0)),
            ),
        ),
        compiler_params=pltpu.CompilerParams(
            dimension_semantics=("arbitrary",),
            vmem_limit_bytes=32 * 1024 * 1024,
        ),
    )(c_disp, motion, *operands)


def kernel(gcn_output, motion, c_disp, h, c, node_positions,
           w_l0, b_l0, w_l1, b_l1, w_fc1, b_fc1, w_fc2, b_fc2, w_fc3, b_fc3):
    motion = motion.astype(jnp.int32)
    gr = jnp.maximum(gcn_output, 0.0).reshape(3, 11, 3, 44)
    h0, c0, part = _run_lstm_pair(c_disp, motion, gr, h, c,
                                  b_l0, w_l0, w_l1)
    out, h_stack, c_stack, cdn = _run_head(
        c_disp, motion, part, b_l1, c, h0, c0,
        w_fc1, b_fc1, w_fc2, b_fc2, w_fc3, b_fc3)

    out = out[:, :_OUT].reshape(_EGO * _EGO, _NCLS)
    new_state = {
        "c_disp": cdn.reshape(2),
        "h": h_stack,
        "c": c_stack,
        "node_positions": node_positions,
    }
    return out, new_state
